# Initial kernel scaffold; baseline (speedup 1.0000x reference)
import jax
import jax.numpy as jnp
from jax.experimental import pallas as pl


def kernel(outputs_l, outputs_c, labels):
    raise NotImplementedError("write your pallas kernel here")


# speedup vs baseline: 2.7971x; 2.7971x over previous
import functools

import jax
import jax.numpy as jnp
from jax.experimental import pallas as pl
from jax.experimental.pallas import tpu as pltpu

RF_CENTERS = 96.5
TH_C = 0.6
IMG_H = 193.0
TH2 = 12.0 * IMG_H
N = 128
K = 1024


def _body(lx_ref, ly_ref, c0_ref, c1_ref, lab_ref, out_ref, min2_ref):
    labf = lab_ref[...]
    px = labf[:, 0:1]
    py = labf[:, 1:2]
    pp = px * px + py * py

    def row(r, _):
        qxr = lx_ref[pl.ds(r, 1), :] * 10.0 + RF_CENTERS
        qyr = ly_ref[pl.ds(r, 1), :] * 10.0 + RF_CENTERS
        qr = jnp.concatenate([qxr, qyr], axis=0)
        g = jax.lax.dot_general(
            labf, qr, (((1,), (0,)), ((), ())),
            preferred_element_type=jnp.float32)
        d = pp - 2.0 * g
        m = jnp.min(d, axis=0, keepdims=True)
        min2_ref[pl.ds(r, 1), :] = m + (qxr * qxr + qyr * qyr)
        return 0

    jax.lax.fori_loop(0, N, row, 0, unroll=8)

    min2 = min2_ref[...]
    c0 = c0_ref[...]
    c1 = c1_ref[...]

    near = min2 < TH2
    far = min2 > TH2
    pos = c1 > TH_C
    neg = c0 > TH_C
    tp = pos & near
    fp = pos & far
    tn = neg & far
    fn = neg & near

    mx = jnp.maximum(c0, c1)
    lse = mx + jnp.log(jnp.exp(c0 - mx) + jnp.exp(c1 - mx))
    v0 = lse - c0
    v1 = lse - c1

    ii = jax.lax.broadcasted_iota(jnp.int32, (N, N), 0)
    jj = jax.lax.broadcasted_iota(jnp.int32, (N, N), 1)
    tri_incl = (ii <= jj).astype(jnp.float32)
    tri_strict = (jj < ii).astype(jnp.float32)

    def mask_stats(mask):
        m32 = mask.astype(jnp.float32)
        csum = jax.lax.dot_general(
            m32, tri_incl, (((1,), (0,)), ((), ())),
            preferred_element_type=jnp.float32)
        tot = csum[:, N - 1:N]
        pref = jax.lax.dot_general(
            tri_strict, tot, (((1,), (0,)), ((), ())),
            preferred_element_type=jnp.float32)
        rank_incl = pref + csum
        count = pref[N - 1, 0] + tot[N - 1, 0]
        return m32, rank_incl, count

    tp32, tp_rank, tp_n = mask_stats(tp)
    fp32, fp_rank, fp_n = mask_stats(fp)
    tn32, tn_rank, tn_n = mask_stats(tn)
    fn32, fn_rank, fn_n = mask_stats(fn)

    min_n = jnp.minimum(jnp.minimum(tp_n, fp_n), jnp.minimum(tn_n, fn_n))
    min_n = jnp.where(min_n == 0.0, 10.0, min_n)

    def term(m32, rank_incl, count, v):
        incl = (m32 > 0.0) & (rank_incl <= min_n)
        s = jnp.sum(jnp.where(incl, v, 0.0))
        n = jnp.minimum(count, min_n)
        return jnp.where(count > 0.0, s / jnp.maximum(n, 1.0), 0.0)

    loss = (term(tp32, tp_rank, tp_n, v1)
            + term(fp32, fp_rank, fp_n, v0)
            + term(tn32, tn_rank, tn_n, v0)
            + term(fn32, fn_rank, fn_n, v1))
    out_ref[0, 0] = loss


@jax.jit
def kernel(outputs_l, outputs_c, labels):
    lx = outputs_l[0, 0]
    ly = outputs_l[0, 1]
    c0 = outputs_c[0, 0]
    c1 = outputs_c[0, 1]
    labf = labels[0].astype(jnp.float32)
    out = pl.pallas_call(
        _body,
        out_shape=jax.ShapeDtypeStruct((1, 1), jnp.float32),
        out_specs=pl.BlockSpec(memory_space=pltpu.SMEM),
        scratch_shapes=[pltpu.VMEM((N, N), jnp.float32)],
    )(lx, ly, c0, c1, labf)
    return out[0, 0]
